# baseline (speedup 1.0000x reference)
import functools

import jax
import jax.numpy as jnp
from jax import lax
from jax.experimental import pallas as pl
from jax.experimental.pallas import tpu as pltpu
from jax.experimental.pallas import tpu_sc as plsc

NU = 10000
NI = 10000
N2 = NU + NI
E = 320000
D_IN = 128
H0 = 64
H1 = 32
NCLS = 5
NB = 2
D_SIDE = 64
FH = 64

E_PAD = 327680
DUMMY = NU
NDEG = 10240

SC_CORES = 2
SC_SUBCORES = 16
SC_TILES = SC_CORES * SC_SUBCORES

_mesh = plsc.VectorSubcoreMesh(
    core_axis_name="c", subcore_axis_name="s",
    num_cores=SC_CORES, num_subcores=SC_SUBCORES)
_sc_params = pltpu.CompilerParams(use_tc_tiling_on_sc=False)


DEGW = 16


def _deg_body(eu2d_hbm, ev2d_hbm, out_hbm, sidx, ones_rows, zbuf, accs):
    c = lax.axis_index("c")
    tid = lax.axis_index("s")
    zrows = NDEG // SC_SUBCORES

    def zrow(i, _):
        zbuf[i, pl.ds(0, 16)] = jnp.zeros((16,), jnp.float32)
        ones_rows[i % 128, pl.ds(0, 16)] = jnp.ones((16,), jnp.float32)
        return 0
    lax.fori_loop(0, zrows, zrow, 0)
    pltpu.sync_copy(zbuf, accs.at[pl.ds(tid * zrows, zrows)])
    plsc.subcore_barrier()

    rows_per_tile = (E_PAD // 128) // SC_SUBCORES
    base = tid * rows_per_tile

    def chunk(it, _):
        roff = base + it * 16

        @pl.when(c == 0)
        def _():
            pltpu.sync_copy(eu2d_hbm.at[pl.ds(roff, 16)], sidx)

        @pl.when(c == 1)
        def _():
            pltpu.sync_copy(ev2d_hbm.at[pl.ds(roff, 16)], sidx)

        for j in range(16):
            pltpu.sync_copy(ones_rows, accs.at[sidx.at[j]], add=True)
        return 0
    lax.fori_loop(0, rows_per_tile // 16, chunk, 0)

    plsc.subcore_barrier()
    pltpu.sync_copy(accs.at[pl.ds(tid * zrows, zrows)],
                    out_hbm.at[c, pl.ds(tid * zrows, zrows)])


def _run_deg(eu2, ev2):
    return pl.kernel(
        _deg_body,
        out_type=jax.ShapeDtypeStruct((2, NDEG, DEGW), jnp.float32),
        mesh=_mesh,
        compiler_params=_sc_params,
        scratch_types=[
            pltpu.VMEM((16, 128), jnp.int32),
            pltpu.VMEM((128, DEGW), jnp.float32),
            pltpu.VMEM((NDEG // SC_SUBCORES, DEGW), jnp.float32),
            pltpu.VMEM_SHARED((NDEG, DEGW), jnp.float32),
        ],
    )(eu2, ev2)


def _dense1_body(catf_ref, cats_ref, wc_ref, w1_ref, b1_ref, degp_ref,
                 table_ref, feat_ref):
    i = pl.program_id(0)
    side = i // 10
    offs = (i % 10) * 1000
    dp = degp_ref[pl.ds(side, 1), pl.ds(offs, 1000), :]
    deg = dp[0, :, 0]
    nsc = lax.rsqrt(jnp.maximum(deg, 1.0))
    x = catf_ref[...]
    sfe = cats_ref[...]
    feat_ref[...] = jnp.maximum(
        jnp.dot(sfe, w1_ref[...], preferred_element_type=jnp.float32)
        + b1_ref[0, :][None, :], 0.0)
    for r in range(NCLS):
        t = jnp.dot(x, wc_ref[r], preferred_element_type=jnp.float32)
        table_ref[r] = t * nsc[:, None]


def _run_dense1(catf, cats, W_conv, W1, b1, degp):
    grid = 20
    return pl.pallas_call(
        _dense1_body,
        grid=(grid,),
        in_specs=[
            pl.BlockSpec((1000, D_IN), lambda i: (i, 0)),
            pl.BlockSpec((1000, D_SIDE), lambda i: (i, 0)),
            pl.BlockSpec((NCLS, D_IN, H0), lambda i: (0, 0, 0)),
            pl.BlockSpec((D_SIDE, FH), lambda i: (0, 0)),
            pl.BlockSpec((1, FH), lambda i: (0, 0)),
            pl.BlockSpec((2, NDEG, DEGW), lambda i: (0, 0, 0)),
        ],
        out_specs=[
            pl.BlockSpec((NCLS, 1000, H0), lambda i: (0, i, 0)),
            pl.BlockSpec((1000, FH), lambda i: (i, 0)),
        ],
        out_shape=[
            jax.ShapeDtypeStruct((NCLS, N2, H0), jnp.float32),
            jax.ShapeDtypeStruct((N2, FH), jnp.float32),
        ],
    )(catf, cats, W_conv, W1, b1, degp)


_MSG_SLOTS = 8
_MSG_BIGS = 4


def _msg_body(table_hbm, gu2d_hbm, gv2d_hbm, eu2d_hbm, ev2d_hbm, out_hbm,
              gidx, sidx, rows, accs, *sems):
    gsems = sems[:_MSG_SLOTS]
    ssems = sems[_MSG_SLOTS:]
    c = lax.axis_index("c")
    tid = lax.axis_index("s")
    per_tile = E_PAD // SC_SUBCORES
    zrows = NDEG // SC_SUBCORES
    big_sz = per_tile // _MSG_BIGS
    hrows = big_sz // 128

    def zrow(i, _):
        for j in range(4):
            rows[i, pl.ds(j * 16, 16)] = jnp.zeros((16,), jnp.float32)
        return 0
    lax.fori_loop(0, zrows, zrow, 0)
    pltpu.sync_copy(rows.at[pl.ds(0, zrows)],
                    accs.at[pl.ds(tid * zrows, zrows)])
    plsc.subcore_barrier()

    for big in range(_MSG_BIGS):
        brow = (tid * per_tile + big * big_sz) // 128

        if big > 0:
            for s in range(_MSG_SLOTS):
                pltpu.make_async_copy(
                    rows.at[pl.ds(s * 128, 128)],
                    accs.at[pl.ds(0, 128)], ssems[s]).wait()

        @pl.when(c == 0)
        def _():
            pltpu.sync_copy(gu2d_hbm.at[pl.ds(brow, hrows)], gidx)
            pltpu.sync_copy(eu2d_hbm.at[pl.ds(brow, hrows)], sidx)

        @pl.when(c == 1)
        def _():
            pltpu.sync_copy(gv2d_hbm.at[pl.ds(brow, hrows)], gidx)
            pltpu.sync_copy(ev2d_hbm.at[pl.ds(brow, hrows)], sidx)

        def body(it, _):
            gds = []
            for s in range(_MSG_SLOTS):
                g = it * _MSG_SLOTS + s
                slot = rows.at[pl.ds(s * 128, 128)]

                @pl.when(it > 0)
                def _(slot=slot, s=s):
                    pltpu.make_async_copy(
                        slot, accs.at[pl.ds(0, 128)], ssems[s]).wait()
                gds.append(pltpu.async_copy(
                    table_hbm.at[gidx.at[g]], slot, gsems[s]))
            for s in range(_MSG_SLOTS):
                g = it * _MSG_SLOTS + s
                slot = rows.at[pl.ds(s * 128, 128)]
                gds[s].wait()
                pltpu.async_copy(slot, accs.at[sidx.at[g]], ssems[s],
                                 add=True)
            return 0
        lax.fori_loop(0, hrows // _MSG_SLOTS, body, 0)

    for s in range(_MSG_SLOTS):
        pltpu.make_async_copy(rows.at[pl.ds(s * 128, 128)],
                              accs.at[pl.ds(0, 128)], ssems[s]).wait()

    plsc.subcore_barrier()
    pltpu.sync_copy(accs.at[pl.ds(tid * zrows, zrows)],
                    out_hbm.at[c, pl.ds(tid * zrows, zrows)])


def _run_msg(table_flat, gu2, gv2, eu2, ev2):
    return pl.kernel(
        _msg_body,
        out_type=jax.ShapeDtypeStruct((2, NDEG, H0), jnp.float32),
        mesh=_mesh,
        compiler_params=_sc_params,
        scratch_types=[
            pltpu.VMEM((E_PAD // SC_SUBCORES // _MSG_BIGS // 128, 128),
                       jnp.int32),
            pltpu.VMEM((E_PAD // SC_SUBCORES // _MSG_BIGS // 128, 128),
                       jnp.int32),
            pltpu.VMEM((_MSG_SLOTS * 128, H0), jnp.float32),
            pltpu.VMEM_SHARED((NDEG, H0), jnp.float32),
        ] + [pltpu.SemaphoreType.DMA] * (2 * _MSG_SLOTS),
    )(table_flat, gu2, gv2, eu2, ev2)


def _dense2_body(s_ref, feat_ref, degp_ref, w2_ref, b2_ref, emb_ref):
    i = pl.program_id(0)
    side = i // 10
    offs = (i % 10) * 1000
    dp = degp_ref[pl.ds(side, 1), pl.ds(offs, 1000), :]
    deg = dp[0, :, 0]
    nsc = lax.rsqrt(jnp.maximum(deg, 1.0))
    gcn = jnp.maximum(s_ref[...] * nsc[:, None], 0.0)
    h = (jnp.dot(gcn, w2_ref[:H0, :], preferred_element_type=jnp.float32)
         + jnp.dot(feat_ref[...], w2_ref[H0:, :],
                   preferred_element_type=jnp.float32)
         + b2_ref[0, :][None, :])
    emb_ref[...] = jnp.maximum(h, 0.0)


def _run_dense2(S, feat, degp, W2, b2):
    return pl.pallas_call(
        _dense2_body,
        grid=(20,),
        in_specs=[
            pl.BlockSpec((1000, H0), lambda i: (i, 0)),
            pl.BlockSpec((1000, FH), lambda i: (i, 0)),
            pl.BlockSpec((2, NDEG, DEGW), lambda i: (0, 0, 0)),
            pl.BlockSpec((H0 + FH, H1), lambda i: (0, 0)),
            pl.BlockSpec((1, H1), lambda i: (0, 0)),
        ],
        out_specs=pl.BlockSpec((1000, H1), lambda i: (i, 0)),
        out_shape=jax.ShapeDtypeStruct((N2, H1), jnp.float32),
    )(S, feat, degp, W2, b2)


_PAIR_UNIT = E_PAD // (SC_SUBCORES * 8)


def _pair_body(emb_hbm, ui_hbm, vi_hbm, ue_hbm, ve_hbm,
               ui_v, vi_v, rows_u, rows_v, gsem, wsem):
    c = lax.axis_index("c")
    s = lax.axis_index("s")

    def run_range(base, units):
        per = units * _PAIR_UNIT
        pltpu.sync_copy(ui_hbm.at[pl.ds(base // 128, per // 128)],
                        ui_v.at[pl.ds(0, per // 128)])
        pltpu.sync_copy(vi_hbm.at[pl.ds(base // 128, per // 128)],
                        vi_v.at[pl.ds(0, per // 128)])

        def chunk(it, _):
            off = base + it * 512
            bank = lax.rem(it, 2)

            @pl.when(it >= 2)
            def _():
                pltpu.make_async_copy(
                    rows_u.at[0], ue_hbm.at[pl.ds(base, 512)], wsem).wait()
                pltpu.make_async_copy(
                    rows_v.at[0], ve_hbm.at[pl.ds(base, 512)], wsem).wait()

            descs = []
            for k in range(4):
                descs.append(pltpu.async_copy(
                    emb_hbm.at[ui_v.at[it * 4 + k]],
                    rows_u.at[bank, pl.ds(k * 128, 128)], gsem))
                descs.append(pltpu.async_copy(
                    emb_hbm.at[vi_v.at[it * 4 + k]],
                    rows_v.at[bank, pl.ds(k * 128, 128)], gsem))
            for d in descs:
                d.wait()
            pltpu.async_copy(rows_u.at[bank], ue_hbm.at[pl.ds(off, 512)],
                             wsem)
            pltpu.async_copy(rows_v.at[bank], ve_hbm.at[pl.ds(off, 512)],
                             wsem)
            return 0
        lax.fori_loop(0, per // 512, chunk, 0)

        for _ in range(2):
            pltpu.make_async_copy(
                rows_u.at[0], ue_hbm.at[pl.ds(base, 512)], wsem).wait()
            pltpu.make_async_copy(
                rows_v.at[0], ve_hbm.at[pl.ds(base, 512)], wsem).wait()

    @pl.when(c == 0)
    def _():
        run_range(s * 7 * _PAIR_UNIT, 7)

    @pl.when(c == 1)
    def _():
        run_range((7 * SC_SUBCORES + s) * _PAIR_UNIT, 1)


def _run_pair(emb, ui2, vi2):
    return pl.kernel(
        _pair_body,
        out_type=(jax.ShapeDtypeStruct((E_PAD, H1), jnp.float32),
                  jax.ShapeDtypeStruct((E_PAD, H1), jnp.float32)),
        mesh=_mesh,
        compiler_params=_sc_params,
        scratch_types=[
            pltpu.VMEM((7 * _PAIR_UNIT // 128, 128), jnp.int32),
            pltpu.VMEM((7 * _PAIR_UNIT // 128, 128), jnp.int32),
            pltpu.VMEM((2, 512, H1), jnp.float32),
            pltpu.VMEM((2, 512, H1), jnp.float32),
            pltpu.SemaphoreType.DMA,
            pltpu.SemaphoreType.DMA,
        ],
    )(emb, ui2, vi2)


_DEC_PK = 128 // H1
_DEC_BR = 1280


def _dec_body(ue_ref, ve_ref, p4_ref, a_ref, out_ref):
    x = ue_ref[...]
    v = ve_ref[...]
    jq = lax.broadcasted_iota(jnp.int32, (128, _DEC_PK), 0) // H1
    qq = lax.broadcasted_iota(jnp.int32, (128, _DEC_PK), 1)
    sel = (jq == qq).astype(jnp.float32)
    z0 = jnp.dot(x, p4_ref[0], preferred_element_type=jnp.float32) * v
    z1 = jnp.dot(x, p4_ref[1], preferred_element_type=jnp.float32) * v
    t0 = jnp.dot(z0, sel, preferred_element_type=jnp.float32)
    t1 = jnp.dot(z1, sel, preferred_element_type=jnp.float32)
    av = a_ref[...]
    ls = [t0 * av[r, 0] + t1 * av[r, 1] for r in range(NCLS)]
    m = ls[0]
    for l in ls[1:]:
        m = jnp.maximum(m, l)
    l20 = jnp.concatenate(ls, axis=1) - jnp.concatenate([m] * NCLS, axis=1)
    e20 = jnp.exp(l20)
    tot = e20[:, 0:4]
    for r in range(1, NCLS):
        tot = tot + e20[:, r * 4:r * 4 + 4]
    inv = 1.0 / tot
    out_ref[...] = e20 * jnp.concatenate([inv] * NCLS, axis=1)


def _run_dec(uer, ver, P4, a):
    return pl.pallas_call(
        _dec_body,
        grid=(E_PAD // (_DEC_BR * _DEC_PK),),
        in_specs=[
            pl.BlockSpec((_DEC_BR, 128), lambda i: (i, 0)),
            pl.BlockSpec((_DEC_BR, 128), lambda i: (i, 0)),
            pl.BlockSpec((NB, 128, 128), lambda i: (0, 0, 0)),
            pl.BlockSpec((NCLS, NB), lambda i: (0, 0)),
        ],
        out_specs=pl.BlockSpec((_DEC_BR, _DEC_PK * NCLS), lambda i: (i, 0)),
        out_shape=jax.ShapeDtypeStruct((E_PAD // _DEC_PK, _DEC_PK * NCLS),
                                       jnp.float32),
    )(uer, ver, P4, a)


def kernel(u_features, v_features, u_features_side, v_features_side,
           W_conv, W1, b1, W2, b2, P, a,
           edge_u, edge_v, edge_class, u_indices, v_indices):
    pad = E_PAD - E
    i32 = jnp.int32
    eu_p = jnp.concatenate([edge_u.astype(i32),
                            jnp.full((pad,), DUMMY, i32)])
    ev_p = jnp.concatenate([edge_v.astype(i32),
                            jnp.full((pad,), DUMMY, i32)])
    ec_p = jnp.concatenate([edge_class.astype(i32), jnp.zeros((pad,), i32)])
    ui_p = jnp.concatenate([u_indices.astype(i32), jnp.zeros((pad,), i32)])
    vi_p = jnp.concatenate([v_indices.astype(i32) + NU, jnp.zeros((pad,), i32)])

    eu2 = eu_p.reshape(E_PAD // 128, 128)
    ev2 = ev_p.reshape(E_PAD // 128, 128)
    ui2 = ui_p.reshape(E_PAD // 128, 128)
    vi2 = vi_p.reshape(E_PAD // 128, 128)
    gu2 = (ec_p * N2 + NU + ev_p).reshape(E_PAD // 128, 128)
    gv2 = (ec_p * N2 + eu_p).reshape(E_PAD // 128, 128)

    degp = _run_deg(eu2, ev2)

    catf = jnp.concatenate([u_features, v_features], axis=0)
    cats = jnp.concatenate([u_features_side, v_features_side], axis=0)
    table, feat = _run_dense1(catf, cats, W_conv, W1,
                              b1.reshape(1, FH), degp)

    Sacc = _run_msg(table.reshape(NCLS * N2, H0), gu2, gv2, eu2, ev2)
    S = Sacc[:, :NU, :].reshape(N2, H0)

    emb = _run_dense2(S, feat, degp, W2, b2.reshape(1, H1))

    ue_g, ve_g = _run_pair(emb, ui2, vi2)

    eye4 = jnp.eye(_DEC_PK, dtype=jnp.float32)
    P4 = jnp.stack([jnp.kron(eye4, P[b]) for b in range(NB)])
    out = _run_dec(ue_g.reshape(E_PAD // _DEC_PK, 128),
                   ve_g.reshape(E_PAD // _DEC_PK, 128), P4, a)
    out = out.reshape(E_PAD // _DEC_PK, NCLS, _DEC_PK)
    return out.transpose(0, 2, 1).reshape(E_PAD, NCLS)[:E]


# speedup vs baseline: 1.3449x; 1.3449x over previous
import functools

import jax
import jax.numpy as jnp
from jax import lax
from jax.experimental import pallas as pl
from jax.experimental.pallas import tpu as pltpu
from jax.experimental.pallas import tpu_sc as plsc

NU = 10000
NI = 10000
N2 = NU + NI
E = 320000
D_IN = 128
H0 = 64
H1 = 32
NCLS = 5
NB = 2
D_SIDE = 64
FH = 64

E_PAD = 327680
DUMMY = NU
NDEG = 10240

SC_CORES = 2
SC_SUBCORES = 16
SC_TILES = SC_CORES * SC_SUBCORES

_mesh = plsc.VectorSubcoreMesh(
    core_axis_name="c", subcore_axis_name="s",
    num_cores=SC_CORES, num_subcores=SC_SUBCORES)
_sc_params = pltpu.CompilerParams(use_tc_tiling_on_sc=False)


DEGW = 16


def _deg_body(eu2d_hbm, ev2d_hbm, out_hbm, sidx, ones_rows, zbuf, accs):
    c = lax.axis_index("c")
    tid = lax.axis_index("s")
    zrows = NDEG // SC_SUBCORES

    def zrow(i, _):
        zbuf[i, pl.ds(0, 16)] = jnp.zeros((16,), jnp.float32)
        ones_rows[i % 128, pl.ds(0, 16)] = jnp.ones((16,), jnp.float32)
        return 0
    lax.fori_loop(0, zrows, zrow, 0)
    pltpu.sync_copy(zbuf, accs.at[pl.ds(tid * zrows, zrows)])
    plsc.subcore_barrier()

    rows_per_tile = (E_PAD // 128) // SC_SUBCORES
    base = tid * rows_per_tile

    def chunk(it, _):
        roff = base + it * 16

        @pl.when(c == 0)
        def _():
            pltpu.sync_copy(eu2d_hbm.at[pl.ds(roff, 16)], sidx)

        @pl.when(c == 1)
        def _():
            pltpu.sync_copy(ev2d_hbm.at[pl.ds(roff, 16)], sidx)

        for j in range(16):
            pltpu.sync_copy(ones_rows, accs.at[sidx.at[j]], add=True)
        return 0
    lax.fori_loop(0, rows_per_tile // 16, chunk, 0)

    plsc.subcore_barrier()
    pltpu.sync_copy(accs.at[pl.ds(tid * zrows, zrows)],
                    out_hbm.at[c, pl.ds(tid * zrows, zrows)])


def _run_deg(eu2, ev2):
    return pl.kernel(
        _deg_body,
        out_type=jax.ShapeDtypeStruct((2, NDEG, DEGW), jnp.float32),
        mesh=_mesh,
        compiler_params=_sc_params,
        scratch_types=[
            pltpu.VMEM((16, 128), jnp.int32),
            pltpu.VMEM((128, DEGW), jnp.float32),
            pltpu.VMEM((NDEG // SC_SUBCORES, DEGW), jnp.float32),
            pltpu.VMEM_SHARED((NDEG, DEGW), jnp.float32),
        ],
    )(eu2, ev2)


def _dense1_body(catf_ref, cats_ref, wc_ref, w1_ref, b1_ref, degp_ref,
                 table_ref, feat_ref):
    i = pl.program_id(0)
    side = i // 10
    offs = (i % 10) * 1000
    dp = degp_ref[pl.ds(side, 1), pl.ds(offs, 1000), :]
    deg = dp[0, :, 0]
    nsc = lax.rsqrt(jnp.maximum(deg, 1.0))
    x = catf_ref[...]
    sfe = cats_ref[...]
    feat_ref[...] = jnp.maximum(
        jnp.dot(sfe, w1_ref[...], preferred_element_type=jnp.float32)
        + b1_ref[0, :][None, :], 0.0)
    for r in range(NCLS):
        t = jnp.dot(x, wc_ref[r], preferred_element_type=jnp.float32)
        table_ref[r] = t * nsc[:, None]


def _run_dense1(catf, cats, W_conv, W1, b1, degp):
    grid = 20
    return pl.pallas_call(
        _dense1_body,
        grid=(grid,),
        in_specs=[
            pl.BlockSpec((1000, D_IN), lambda i: (i, 0)),
            pl.BlockSpec((1000, D_SIDE), lambda i: (i, 0)),
            pl.BlockSpec((NCLS, D_IN, H0), lambda i: (0, 0, 0)),
            pl.BlockSpec((D_SIDE, FH), lambda i: (0, 0)),
            pl.BlockSpec((1, FH), lambda i: (0, 0)),
            pl.BlockSpec((2, NDEG, DEGW), lambda i: (0, 0, 0)),
        ],
        out_specs=[
            pl.BlockSpec((NCLS, 1000, H0), lambda i: (0, i, 0)),
            pl.BlockSpec((1000, FH), lambda i: (i, 0)),
        ],
        out_shape=[
            jax.ShapeDtypeStruct((NCLS, N2, H0), jnp.float32),
            jax.ShapeDtypeStruct((N2, FH), jnp.float32),
        ],
    )(catf, cats, W_conv, W1, b1, degp)


_MSG_SLOTS = 8
_MSG_BIGS = 4


def _msg_body(table_hbm, gu2d_hbm, gv2d_hbm, eu2d_hbm, ev2d_hbm, out_hbm,
              gidx, sidx, rows, accs, *sems):
    gsems = sems[:_MSG_SLOTS]
    ssems = sems[_MSG_SLOTS:]
    c = lax.axis_index("c")
    tid = lax.axis_index("s")
    per_tile = E_PAD // SC_SUBCORES
    zrows = NDEG // SC_SUBCORES
    big_sz = per_tile // _MSG_BIGS
    hrows = big_sz // 128

    def zrow(i, _):
        for j in range(4):
            rows[i, pl.ds(j * 16, 16)] = jnp.zeros((16,), jnp.float32)
        return 0
    lax.fori_loop(0, zrows, zrow, 0)
    pltpu.sync_copy(rows.at[pl.ds(0, zrows)],
                    accs.at[pl.ds(tid * zrows, zrows)])
    plsc.subcore_barrier()

    for big in range(_MSG_BIGS):
        brow = (tid * per_tile + big * big_sz) // 128

        if big > 0:
            for s in range(_MSG_SLOTS):
                pltpu.make_async_copy(
                    rows.at[pl.ds(s * 128, 128)],
                    accs.at[pl.ds(0, 128)], ssems[s]).wait()

        @pl.when(c == 0)
        def _():
            pltpu.sync_copy(gu2d_hbm.at[pl.ds(brow, hrows)], gidx)
            pltpu.sync_copy(eu2d_hbm.at[pl.ds(brow, hrows)], sidx)

        @pl.when(c == 1)
        def _():
            pltpu.sync_copy(gv2d_hbm.at[pl.ds(brow, hrows)], gidx)
            pltpu.sync_copy(ev2d_hbm.at[pl.ds(brow, hrows)], sidx)

        def body(it, _):
            gds = []
            for s in range(_MSG_SLOTS):
                g = it * _MSG_SLOTS + s
                slot = rows.at[pl.ds(s * 128, 128)]

                @pl.when(it > 0)
                def _(slot=slot, s=s):
                    pltpu.make_async_copy(
                        slot, accs.at[pl.ds(0, 128)], ssems[s]).wait()
                gds.append(pltpu.async_copy(
                    table_hbm.at[gidx.at[g]], slot, gsems[s]))
            for s in range(_MSG_SLOTS):
                g = it * _MSG_SLOTS + s
                slot = rows.at[pl.ds(s * 128, 128)]
                gds[s].wait()
                pltpu.async_copy(slot, accs.at[sidx.at[g]], ssems[s],
                                 add=True)
            return 0
        lax.fori_loop(0, hrows // _MSG_SLOTS, body, 0)

    for s in range(_MSG_SLOTS):
        pltpu.make_async_copy(rows.at[pl.ds(s * 128, 128)],
                              accs.at[pl.ds(0, 128)], ssems[s]).wait()

    plsc.subcore_barrier()
    pltpu.sync_copy(accs.at[pl.ds(tid * zrows, zrows)],
                    out_hbm.at[c, pl.ds(tid * zrows, zrows)])


def _run_msg(table_flat, gu2, gv2, eu2, ev2):
    return pl.kernel(
        _msg_body,
        out_type=jax.ShapeDtypeStruct((2, NDEG, H0), jnp.float32),
        mesh=_mesh,
        compiler_params=_sc_params,
        scratch_types=[
            pltpu.VMEM((E_PAD // SC_SUBCORES // _MSG_BIGS // 128, 128),
                       jnp.int32),
            pltpu.VMEM((E_PAD // SC_SUBCORES // _MSG_BIGS // 128, 128),
                       jnp.int32),
            pltpu.VMEM((_MSG_SLOTS * 128, H0), jnp.float32),
            pltpu.VMEM_SHARED((NDEG, H0), jnp.float32),
        ] + [pltpu.SemaphoreType.DMA] * (2 * _MSG_SLOTS),
    )(table_flat, gu2, gv2, eu2, ev2)


def _dense2_body(s_ref, feat_ref, degp_ref, w2_ref, b2_ref, emb_ref):
    i = pl.program_id(0)
    side = i // 10
    offs = (i % 10) * 1000
    dp = degp_ref[pl.ds(side, 1), pl.ds(offs, 1000), :]
    deg = dp[0, :, 0]
    nsc = lax.rsqrt(jnp.maximum(deg, 1.0))
    gcn = jnp.maximum(s_ref[...] * nsc[:, None], 0.0)
    h = (jnp.dot(gcn, w2_ref[:H0, :], preferred_element_type=jnp.float32)
         + jnp.dot(feat_ref[...], w2_ref[H0:, :],
                   preferred_element_type=jnp.float32)
         + b2_ref[0, :][None, :])
    emb_ref[...] = jnp.maximum(h, 0.0)


def _run_dense2(S, feat, degp, W2, b2):
    return pl.pallas_call(
        _dense2_body,
        grid=(20,),
        in_specs=[
            pl.BlockSpec((1000, H0), lambda i: (i, 0)),
            pl.BlockSpec((1000, FH), lambda i: (i, 0)),
            pl.BlockSpec((2, NDEG, DEGW), lambda i: (0, 0, 0)),
            pl.BlockSpec((H0 + FH, H1), lambda i: (0, 0)),
            pl.BlockSpec((1, H1), lambda i: (0, 0)),
        ],
        out_specs=pl.BlockSpec((1000, H1), lambda i: (i, 0)),
        out_shape=jax.ShapeDtypeStruct((N2, H1), jnp.float32),
    )(S, feat, degp, W2, b2)


_PAIR_UNIT = E_PAD // (SC_SUBCORES * 4)


def _pair_body(emb_hbm, ui_hbm, vi_hbm, ue_hbm, ve_hbm,
               ui_v, vi_v, rows_u, rows_v, gsem, wsem):
    c = lax.axis_index("c")
    s = lax.axis_index("s")

    def run_range(base, units):
        per = units * _PAIR_UNIT
        pltpu.sync_copy(ui_hbm.at[pl.ds(base // 128, per // 128)],
                        ui_v.at[pl.ds(0, per // 128)])
        pltpu.sync_copy(vi_hbm.at[pl.ds(base // 128, per // 128)],
                        vi_v.at[pl.ds(0, per // 128)])

        def chunk(it, _):
            off = base + it * 512
            bank = lax.rem(it, 2)

            @pl.when(it >= 2)
            def _():
                pltpu.make_async_copy(
                    rows_u.at[0], ue_hbm.at[pl.ds(base, 512)], wsem).wait()
                pltpu.make_async_copy(
                    rows_v.at[0], ve_hbm.at[pl.ds(base, 512)], wsem).wait()

            descs = []
            for k in range(4):
                descs.append(pltpu.async_copy(
                    emb_hbm.at[ui_v.at[it * 4 + k]],
                    rows_u.at[bank, pl.ds(k * 128, 128)], gsem))
                descs.append(pltpu.async_copy(
                    emb_hbm.at[vi_v.at[it * 4 + k]],
                    rows_v.at[bank, pl.ds(k * 128, 128)], gsem))
            for d in descs:
                d.wait()
            pltpu.async_copy(rows_u.at[bank], ue_hbm.at[pl.ds(off, 512)],
                             wsem)
            pltpu.async_copy(rows_v.at[bank], ve_hbm.at[pl.ds(off, 512)],
                             wsem)
            return 0
        lax.fori_loop(0, per // 512, chunk, 0)

        for _ in range(2):
            pltpu.make_async_copy(
                rows_u.at[0], ue_hbm.at[pl.ds(base, 512)], wsem).wait()
            pltpu.make_async_copy(
                rows_v.at[0], ve_hbm.at[pl.ds(base, 512)], wsem).wait()

    @pl.when(c == 0)
    def _():
        run_range(s * 3 * _PAIR_UNIT, 3)

    @pl.when(c == 1)
    def _():
        run_range((3 * SC_SUBCORES + s) * _PAIR_UNIT, 1)


def _run_pair(emb, ui2, vi2):
    return pl.kernel(
        _pair_body,
        out_type=(jax.ShapeDtypeStruct((E_PAD, H1), jnp.float32),
                  jax.ShapeDtypeStruct((E_PAD, H1), jnp.float32)),
        mesh=_mesh,
        compiler_params=_sc_params,
        scratch_types=[
            pltpu.VMEM((3 * _PAIR_UNIT // 128, 128), jnp.int32),
            pltpu.VMEM((3 * _PAIR_UNIT // 128, 128), jnp.int32),
            pltpu.VMEM((2, 512, H1), jnp.float32),
            pltpu.VMEM((2, 512, H1), jnp.float32),
            pltpu.SemaphoreType.DMA,
            pltpu.SemaphoreType.DMA,
        ],
    )(emb, ui2, vi2)


_DEC_PK = 128 // H1
_DEC_BR = 1280


def _dec_body(ue_ref, ve_ref, p4_ref, a_ref, out_ref):
    x = ue_ref[...]
    v = ve_ref[...]
    jq = lax.broadcasted_iota(jnp.int32, (128, _DEC_PK), 0) // H1
    qq = lax.broadcasted_iota(jnp.int32, (128, _DEC_PK), 1)
    sel = (jq == qq).astype(jnp.float32)
    z0 = jnp.dot(x, p4_ref[0], preferred_element_type=jnp.float32) * v
    z1 = jnp.dot(x, p4_ref[1], preferred_element_type=jnp.float32) * v
    t0 = jnp.dot(z0, sel, preferred_element_type=jnp.float32)
    t1 = jnp.dot(z1, sel, preferred_element_type=jnp.float32)
    av = a_ref[...]
    ls = [t0 * av[r, 0] + t1 * av[r, 1] for r in range(NCLS)]
    m = ls[0]
    for l in ls[1:]:
        m = jnp.maximum(m, l)
    exs = [jnp.exp(l - m) for l in ls]
    tot = exs[0]
    for e in exs[1:]:
        tot = tot + e
    inv = 1.0 / tot
    cols = []
    for q in range(_DEC_PK):
        for r in range(NCLS):
            cols.append((exs[r] * inv)[:, q:q + 1])
    out_ref[...] = jnp.concatenate(cols, axis=1)


def _run_dec(uer, ver, P4, a):
    return pl.pallas_call(
        _dec_body,
        grid=(E_PAD // (_DEC_BR * _DEC_PK),),
        in_specs=[
            pl.BlockSpec((_DEC_BR, 128), lambda i: (i, 0)),
            pl.BlockSpec((_DEC_BR, 128), lambda i: (i, 0)),
            pl.BlockSpec((NB, 128, 128), lambda i: (0, 0, 0)),
            pl.BlockSpec((NCLS, NB), lambda i: (0, 0)),
        ],
        out_specs=pl.BlockSpec((_DEC_BR, _DEC_PK * NCLS), lambda i: (i, 0)),
        out_shape=jax.ShapeDtypeStruct((E_PAD // _DEC_PK, _DEC_PK * NCLS),
                                       jnp.float32),
    )(uer, ver, P4, a)


def kernel(u_features, v_features, u_features_side, v_features_side,
           W_conv, W1, b1, W2, b2, P, a,
           edge_u, edge_v, edge_class, u_indices, v_indices):
    pad = E_PAD - E
    i32 = jnp.int32
    eu_p = jnp.concatenate([edge_u.astype(i32),
                            jnp.full((pad,), DUMMY, i32)])
    ev_p = jnp.concatenate([edge_v.astype(i32),
                            jnp.full((pad,), DUMMY, i32)])
    ec_p = jnp.concatenate([edge_class.astype(i32), jnp.zeros((pad,), i32)])
    ui_p = jnp.concatenate([u_indices.astype(i32), jnp.zeros((pad,), i32)])
    vi_p = jnp.concatenate([v_indices.astype(i32) + NU, jnp.zeros((pad,), i32)])

    eu2 = eu_p.reshape(E_PAD // 128, 128)
    ev2 = ev_p.reshape(E_PAD // 128, 128)
    ui2 = ui_p.reshape(E_PAD // 128, 128)
    vi2 = vi_p.reshape(E_PAD // 128, 128)
    gu2 = (ec_p * N2 + NU + ev_p).reshape(E_PAD // 128, 128)
    gv2 = (ec_p * N2 + eu_p).reshape(E_PAD // 128, 128)

    degp = _run_deg(eu2, ev2)

    catf = jnp.concatenate([u_features, v_features], axis=0)
    cats = jnp.concatenate([u_features_side, v_features_side], axis=0)
    table, feat = _run_dense1(catf, cats, W_conv, W1,
                              b1.reshape(1, FH), degp)

    Sacc = _run_msg(table.reshape(NCLS * N2, H0), gu2, gv2, eu2, ev2)
    S = Sacc[:, :NU, :].reshape(N2, H0)

    emb = _run_dense2(S, feat, degp, W2, b2.reshape(1, H1))

    ue_g, ve_g = _run_pair(emb, ui2, vi2)

    eye4 = jnp.eye(_DEC_PK, dtype=jnp.float32)
    P4 = jnp.stack([jnp.kron(eye4, P[b]) for b in range(NB)])
    out = _run_dec(ue_g.reshape(E_PAD // _DEC_PK, 128),
                   ve_g.reshape(E_PAD // _DEC_PK, 128), P4, a)
    return out.reshape(E_PAD, NCLS)[:E]
